# Initial kernel scaffold; baseline (speedup 1.0000x reference)
import functools

import jax
import jax.numpy as jnp
import numpy as np
from jax import lax
from jax.experimental import pallas as pl
from jax.experimental.pallas import tpu as pltpu
from jax.experimental.pallas import tpu_sc as plsc

_FM = 14
_STR = 32
_RATIOS = [[4, 4], [3, 5], [5, 3], [6, 6], [5, 7], [7, 5], [8, 8], [6, 10],
           [10, 6], [7, 9], [9, 7]]
_WN = [(_FM - r[0] + 1) * (_FM - r[1] + 1) for r in _RATIOS]
_G_SIZES = [sum(_WN[:3]), sum(_WN[3:6]), sum(_WN[6:])]
_G_PAD = [368, 256, 240]
_G_BASE = [0, 368, 624]
_G_UBASE = [0, 361, 602]
_G_CHUNKS = [23, 16, 15]
_TOT = sum(_G_PAD)
_NKEEP = [2, 3, 2]
_NEG = np.float32(-1e30)


def _np_tables():
    coords = []
    for r in _RATIOS:
        col_num = _FM - r[1] + 1
        row_num = _FM - r[0] + 1
        idx = np.arange(row_num * col_num)
        x_ind = idx // col_num
        y_ind = idx % col_num
        x_lt = x_ind * _STR - 1
        y_lt = y_ind * _STR - 1
        x_rb = x_lt + r[0] * _STR
        y_rb = y_lt + r[1] * _STR
        x_lt = np.maximum(x_lt, 0)
        y_lt = np.maximum(y_lt, 0)
        coords.append(np.stack([x_lt, y_lt, x_rb, y_rb], 1))
    coords = np.concatenate(coords, 0).astype(np.float32)

    pad_pos = np.zeros(837, dtype=np.int64)
    for g in range(3):
        u0, n = _G_UBASE[g], _G_SIZES[g]
        pad_pos[u0:u0 + n] = _G_BASE[g] + np.arange(n)

    wf = np.zeros((_FM * _FM, _TOT), dtype=np.float32)
    u = 0
    for r in _RATIOS:
        kh, kw = r
        col_num = _FM - kw + 1
        row_num = _FM - kh + 1
        for w in range(row_num * col_num):
            xi, yi = w // col_num, w % col_num
            p = pad_pos[u + w]
            for i in range(xi, xi + kh):
                wf[i * _FM + yi:i * _FM + yi + kw, p] = 1.0 / (kh * kw)
        u += row_num * col_num
    wr = wf.reshape(_FM, _FM, _TOT)

    bias = np.zeros((1, _TOT), dtype=np.float32)
    cpad = np.zeros((5, _TOT), dtype=np.float32)
    real = np.zeros(_TOT, dtype=bool)
    real[pad_pos] = True
    bias[0, ~real] = _NEG
    areas = (coords[:, 2] - coords[:, 0]) * (coords[:, 3] - coords[:, 1])
    for k in range(4):
        cpad[k, pad_pos] = coords[:, k]
    cpad[4, pad_pos] = areas
    return wr, bias, cpad


_WR, _BIAS, _CPAD = _np_tables()

_CBLK = 256
_NCB = 2048 // _CBLK


def _tc_body(x_ref, wr_ref, bias_ref, out_ref, acc_ref):
    c = pl.program_id(1)

    @pl.when(c == 0)
    def _():
        acc_ref[...] = jnp.zeros_like(acc_ref)

    z = jnp.zeros((_FM, _FM), jnp.float32)

    def body(i, accs):
        a0, a1, a2, a3 = accs
        b = i * 4
        return (a0 + x_ref[0, b], a1 + x_ref[0, b + 1],
                a2 + x_ref[0, b + 2], a3 + x_ref[0, b + 3])

    a0, a1, a2, a3 = lax.fori_loop(0, _CBLK // 4, body, (z, z, z, z))
    acc_ref[...] = acc_ref[...] + ((a0 + a1) + (a2 + a3))

    @pl.when(c == _NCB - 1)
    def _():
        a = acc_ref[...]
        total = bias_ref[...]
        for i in range(_FM):
            total = total + jnp.dot(a[i:i + 1, :], wr_ref[i],
                                    preferred_element_type=jnp.float32)
        out_ref[...] = total


def _tc_scores(x, wr, bias, interpret=False):
    return pl.pallas_call(
        _tc_body,
        grid=(x.shape[0], _NCB),
        in_specs=[
            pl.BlockSpec((1, _CBLK, _FM, _FM), lambda b, c: (b, c, 0, 0)),
            pl.BlockSpec((_FM, _FM, _TOT), lambda b, c: (0, 0, 0)),
            pl.BlockSpec((1, _TOT), lambda b, c: (0, 0)),
        ],
        out_specs=pl.BlockSpec((1, _TOT), lambda b, c: (b, 0)),
        out_shape=jax.ShapeDtypeStruct((x.shape[0], _TOT), jnp.float32),
        scratch_shapes=[pltpu.VMEM((_FM, _FM), jnp.float32)],
        compiler_params=pltpu.CompilerParams(
            dimension_semantics=("arbitrary", "arbitrary")),
        interpret=interpret,
    )(x, wr, bias)


_PICKS = []
for _g in range(3):
    for _ in range(_NKEEP[_g]):
        _PICKS.append((_g, _G_BASE[_g], _G_UBASE[_g], _G_CHUNKS[_g]))


def _nms_body(scores_hbm, x1_hbm, y1_hbm, x2_hbm, y2_hbm, ar_hbm, thr_hbm,
              idx_out, scr_out,
              s_v, x1_v, y1_v, x2_v, y2_v, ar_v, thr_v, ib_v, sb_v):
    wid = lax.axis_index("s") * 2 + lax.axis_index("c")
    pltpu.sync_copy(x1_hbm, x1_v)
    pltpu.sync_copy(y1_hbm, y1_v)
    pltpu.sync_copy(x2_hbm, x2_v)
    pltpu.sync_copy(y2_hbm, y2_v)
    pltpu.sync_copy(ar_hbm, ar_v)
    lanes = lax.broadcasted_iota(jnp.int32, (16,), 0)

    for smp in range(2):
        b = wid * 2 + smp
        pltpu.sync_copy(scores_hbm.at[b], s_v)
        picks_i = jnp.zeros((16,), jnp.int32)
        picks_s = jnp.zeros((16,), jnp.float32)
        for k, (g, gb, ub, nch) in enumerate(_PICKS):
            def amax_body(cc, carry, gb=gb):
                bv, bc = carry
                v = s_v[pl.ds(gb + cc * 16, 16)]
                better = v > bv
                return (jnp.where(better, v, bv),
                        jnp.where(better, cc, bc))

            bv0 = s_v[pl.ds(gb, 16)]
            bc0 = jnp.zeros((16,), jnp.int32)
            bv, bc = lax.fori_loop(1, nch, amax_body, (bv0, bc0))
            m = jnp.max(bv)
            cand = bc * 16 + lanes
            iloc = jnp.min(jnp.where(bv == m, cand, jnp.int32(1 << 30)))
            picks_i = jnp.where(lanes == k, ub + iloc, picks_i)
            picks_s = jnp.where(lanes == k, m, picks_s)

            ipad = gb + iloc
            idxv = jnp.zeros((16,), jnp.int32) + ipad
            px1 = plsc.load_gather(x1_v, [idxv])
            py1 = plsc.load_gather(y1_v, [idxv])
            px2 = plsc.load_gather(x2_v, [idxv])
            py2 = plsc.load_gather(y2_v, [idxv])
            par = plsc.load_gather(ar_v, [idxv])
            pltpu.sync_copy(thr_hbm.at[g], thr_v)
            thrv = thr_v[...]

            def sup_body(cc, _, gb=gb, px1=px1, py1=py1, px2=px2, py2=py2,
                         par=par, thrv=thrv, ipad=ipad):
                off = gb + cc * 16
                sw = s_v[pl.ds(off, 16)]
                xx1 = jnp.maximum(x1_v[pl.ds(off, 16)], px1)
                yy1 = jnp.maximum(y1_v[pl.ds(off, 16)], py1)
                xx2 = jnp.minimum(x2_v[pl.ds(off, 16)], px2)
                yy2 = jnp.minimum(y2_v[pl.ds(off, 16)], py2)
                inter = (jnp.maximum(xx2 - xx1, 0.0)
                         * jnp.maximum(yy2 - yy1, 0.0))
                iou = inter / (ar_v[pl.ds(off, 16)] + par - inter)
                kill = (iou > thrv) | ((off + lanes) == ipad)
                s_v[pl.ds(off, 16)] = jnp.where(kill, _NEG, sw)
                return 0

            lax.fori_loop(0, nch, sup_body, 0)

        ib_v[...] = picks_i
        sb_v[...] = picks_s
        pltpu.sync_copy(ib_v, idx_out.at[b])
        pltpu.sync_copy(sb_v, scr_out.at[b])


def _make_nms(interpret=False):
    mesh = plsc.VectorSubcoreMesh(core_axis_name="c", subcore_axis_name="s",
                                  num_cores=2, num_subcores=16)
    return pl.kernel(
        _nms_body,
        out_type=(jax.ShapeDtypeStruct((64, 16), jnp.int32),
                  jax.ShapeDtypeStruct((64, 16), jnp.float32)),
        mesh=mesh,
        scratch_types=[pltpu.VMEM((_TOT,), jnp.float32)] * 6
        + [pltpu.VMEM((16,), jnp.float32),
           pltpu.VMEM((16,), jnp.int32),
           pltpu.VMEM((16,), jnp.float32)],
        interpret=interpret,
    )


def _impl(proposalN, x, ratios, window_nums_sum, N_list, iou_threshs,
          interpret=False):
    x = jnp.asarray(x, jnp.float32)
    wr = jnp.asarray(_WR)
    bias = jnp.asarray(_BIAS)
    scores_pad = _tc_scores(x, wr, bias, interpret)
    thr = jnp.broadcast_to(
        jnp.asarray(iou_threshs, jnp.float32)[:, None], (3, 16))
    cpad = jnp.asarray(_CPAD)
    idx16, scr16 = _make_nms(interpret)(
        scores_pad, cpad[0], cpad[1], cpad[2], cpad[3], cpad[4], thr)
    indices = idx16[:, :7]
    pscores = scr16[:, :7]
    wscores = jnp.concatenate(
        [scores_pad[:, _G_BASE[g]:_G_BASE[g] + _G_SIZES[g]] for g in range(3)],
        axis=1)
    return (indices, pscores, wscores)


def kernel(proposalN, x, ratios, window_nums_sum, N_list, iou_threshs):
    return _impl(proposalN, x, ratios, window_nums_sum, N_list, iou_threshs)


# speedup vs baseline: 13.7279x; 13.7279x over previous
import functools

import jax
import jax.numpy as jnp
import numpy as np
from jax import lax
from jax.experimental import pallas as pl
from jax.experimental.pallas import tpu as pltpu
from jax.experimental.pallas import tpu_sc as plsc

_FM = 14
_STR = 32
_RATIOS = [[4, 4], [3, 5], [5, 3], [6, 6], [5, 7], [7, 5], [8, 8], [6, 10],
           [10, 6], [7, 9], [9, 7]]
_WN = [(_FM - r[0] + 1) * (_FM - r[1] + 1) for r in _RATIOS]
_G_SIZES = [sum(_WN[:3]), sum(_WN[3:6]), sum(_WN[6:])]
_G_PAD = [368, 256, 272]
_G_BASE = [0, 368, 624]
_G_UBASE = [0, 361, 602]
_G_CHUNKS = [23, 16, 17]
_TOT = sum(_G_PAD)
_NKEEP = [2, 3, 2]
_NEG = np.float32(-1e30)


def _np_tables():
    coords = []
    for r in _RATIOS:
        col_num = _FM - r[1] + 1
        row_num = _FM - r[0] + 1
        idx = np.arange(row_num * col_num)
        x_ind = idx // col_num
        y_ind = idx % col_num
        x_lt = x_ind * _STR - 1
        y_lt = y_ind * _STR - 1
        x_rb = x_lt + r[0] * _STR
        y_rb = y_lt + r[1] * _STR
        x_lt = np.maximum(x_lt, 0)
        y_lt = np.maximum(y_lt, 0)
        coords.append(np.stack([x_lt, y_lt, x_rb, y_rb], 1))
    coords = np.concatenate(coords, 0).astype(np.float32)

    pad_pos = np.zeros(837, dtype=np.int64)
    for g in range(3):
        u0, n = _G_UBASE[g], _G_SIZES[g]
        pad_pos[u0:u0 + n] = _G_BASE[g] + np.arange(n)

    wf = np.zeros((_FM * _FM, _TOT), dtype=np.float32)
    u = 0
    for r in _RATIOS:
        kh, kw = r
        col_num = _FM - kw + 1
        row_num = _FM - kh + 1
        for w in range(row_num * col_num):
            xi, yi = w // col_num, w % col_num
            p = pad_pos[u + w]
            for i in range(xi, xi + kh):
                wf[i * _FM + yi:i * _FM + yi + kw, p] = 1.0 / (kh * kw)
        u += row_num * col_num
    wr = wf.reshape(_FM, _FM, _TOT)

    bias = np.zeros((1, _TOT), dtype=np.float32)
    cpad = np.zeros((5, _TOT), dtype=np.float32)
    real = np.zeros(_TOT, dtype=bool)
    real[pad_pos] = True
    bias[0, ~real] = _NEG
    areas = (coords[:, 2] - coords[:, 0]) * (coords[:, 3] - coords[:, 1])
    for k in range(4):
        cpad[k, pad_pos] = coords[:, k]
    cpad[4, pad_pos] = areas
    return wr, bias, cpad


_WR, _BIAS, _CPAD = _np_tables()

_WBLK = 14


def _tc_body(xt_ref, wf_ref, bias_ref, out_ref, acc_ref):
    h = pl.program_id(0)

    @pl.when(h == 0)
    def _():
        acc_ref[...] = jnp.zeros_like(acc_ref)

    acc = acc_ref[...]
    wtile = wf_ref[0, 0]
    for w in range(_WBLK):
        col = jnp.sum(xt_ref[0, w], axis=1, keepdims=True)
        acc = acc + col * wtile[w:w + 1, :]
    acc_ref[...] = acc

    @pl.when(h == _FM * (_FM // _WBLK) - 1)
    def _():
        out_ref[...] = acc_ref[...] + bias_ref[...]


def _tc_scores(x, wf3, bias, interpret=False):
    b, c = x.shape[0], x.shape[1]
    xt = jnp.transpose(x, (2, 3, 0, 1))
    return pl.pallas_call(
        _tc_body,
        grid=(_FM * (_FM // _WBLK),),
        in_specs=[
            pl.BlockSpec((1, _WBLK, b, c),
                         lambda h: (h // (_FM // _WBLK), h % (_FM // _WBLK),
                                    0, 0)),
            pl.BlockSpec((1, 1, _WBLK, _TOT),
                         lambda h: (h // (_FM // _WBLK), h % (_FM // _WBLK),
                                    0, 0)),
            pl.BlockSpec((1, _TOT), lambda h: (0, 0)),
        ],
        out_specs=pl.BlockSpec((b, _TOT), lambda h: (0, 0)),
        out_shape=jax.ShapeDtypeStruct((b, _TOT), jnp.float32),
        scratch_shapes=[pltpu.VMEM((b, _TOT), jnp.float32)],
        compiler_params=pltpu.CompilerParams(
            dimension_semantics=("arbitrary",)),
        interpret=interpret,
    )(xt, wf3.reshape(_FM, _FM // _WBLK, _WBLK, _TOT), bias)


_GROUPS = [(g, _G_BASE[g], _G_UBASE[g], _G_CHUNKS[g], _NKEEP[g])
           for g in range(3)]


def _nms_body(scores_hbm, tb_hbm, out_hbm,
              s_v, s2_v, tb_v, ob_v, sem):
    wid = lax.axis_index("s") * 2 + lax.axis_index("c")
    b0 = wid * 2
    cp2 = pltpu.async_copy(scores_hbm.at[pl.ds((b0 + 1) * _TOT, _TOT)],
                           s2_v, sem)
    pltpu.sync_copy(tb_hbm, tb_v)
    pltpu.sync_copy(scores_hbm.at[pl.ds(b0 * _TOT, _TOT)], s_v)
    lanes = lax.broadcasted_iota(jnp.int32, (16,), 0)
    negv = jnp.full((16,), _NEG, jnp.float32)
    bc0 = jnp.zeros((16,), jnp.int32)

    for smp in range(2):
        sv = s_v if smp == 0 else s2_v
        if smp == 1:
            cp2.wait()
        picks = jnp.zeros((16,), jnp.float32)
        k = 0
        for g, gb, ub, nch, npick in _GROUPS:
            thrv = tb_v[pl.ds(5 * _TOT + g * 16, 16)]

            def amax_body(cc, carry, sv=sv, gb=gb):
                bv, bc = carry
                v = sv[pl.ds(gb + cc * 16, 16)]
                better = v > bv
                return (jnp.where(better, v, bv),
                        jnp.where(better, cc, bc))

            bv, bc = lax.fori_loop(1, nch, amax_body,
                                   (sv[pl.ds(gb, 16)], bc0), unroll=2)
            for j in range(npick):
                m = jnp.max(bv)
                cand = bc * 16 + lanes
                iloc = jnp.min(jnp.where(bv == m, cand, jnp.int32(1 << 30)))
                picks = jnp.where(lanes == k, (ub + iloc).astype(jnp.float32),
                                  picks)
                picks = jnp.where(lanes == k + 8, m, picks)
                k += 1
                if j == npick - 1:
                    break
                ipad = gb + iloc
                idxv = bc0 + ipad
                px1 = plsc.load_gather(tb_v, [idxv])
                py1 = plsc.load_gather(tb_v, [idxv + _TOT])
                px2 = plsc.load_gather(tb_v, [idxv + 2 * _TOT])
                py2 = plsc.load_gather(tb_v, [idxv + 3 * _TOT])
                par = plsc.load_gather(tb_v, [idxv + 4 * _TOT])
                tpar = thrv * par

                def fused_body(cc, carry, sv=sv, gb=gb, px1=px1, py1=py1,
                               px2=px2, py2=py2, tpar=tpar, thrv=thrv,
                               ipad=ipad):
                    bv, bc = carry
                    off = gb + cc * 16
                    sw = sv[pl.ds(off, 16)]
                    xx1 = jnp.maximum(tb_v[pl.ds(off, 16)], px1)
                    yy1 = jnp.maximum(tb_v[pl.ds(_TOT + off, 16)], py1)
                    xx2 = jnp.minimum(tb_v[pl.ds(2 * _TOT + off, 16)], px2)
                    yy2 = jnp.minimum(tb_v[pl.ds(3 * _TOT + off, 16)], py2)
                    inter = (jnp.maximum(xx2 - xx1, 0.0)
                             * jnp.maximum(yy2 - yy1, 0.0))
                    kill = (inter + thrv * inter
                            > thrv * tb_v[pl.ds(4 * _TOT + off, 16)] + tpar)
                    sw = jnp.where(kill, _NEG, sw)
                    sv[pl.ds(off, 16)] = sw
                    better = sw > bv
                    return (jnp.where(better, sw, bv),
                            jnp.where(better, cc, bc))

                bv, bc = lax.fori_loop(0, nch, fused_body, (negv, bc0),
                                       unroll=2)

        ob_v[pl.ds(smp * 16, 16)] = picks

    pltpu.sync_copy(ob_v, out_hbm.at[pl.ds(b0 * 16, 32)])


def _make_nms(interpret=False):
    mesh = plsc.VectorSubcoreMesh(core_axis_name="c", subcore_axis_name="s",
                                  num_cores=2, num_subcores=16)
    return pl.kernel(
        _nms_body,
        out_type=jax.ShapeDtypeStruct((64 * 16,), jnp.float32),
        mesh=mesh,
        scratch_types=[pltpu.VMEM((_TOT,), jnp.float32)] * 2
        + [pltpu.VMEM((5 * _TOT + 48,), jnp.float32),
           pltpu.VMEM((32,), jnp.float32),
           pltpu.SemaphoreType.DMA],
        compiler_params=pltpu.CompilerParams(
            needs_layout_passes=False, disable_bounds_checks=True,
            disable_semaphore_checks=True),
        interpret=interpret,
    )


def _impl(proposalN, x, ratios, window_nums_sum, N_list, iou_threshs,
          interpret=False):
    x = jnp.asarray(x, jnp.float32)
    wr = jnp.asarray(_WR)
    bias = jnp.asarray(_BIAS)
    scores_pad = _tc_scores(x, wr, bias, interpret)
    thr = jnp.broadcast_to(
        jnp.asarray(iou_threshs, jnp.float32)[:, None], (3, 16)).reshape(-1)
    tables = jnp.concatenate([jnp.asarray(_CPAD).reshape(-1), thr])
    out16 = _make_nms(interpret)(scores_pad.reshape(-1), tables)
    out16 = out16.reshape(64, 16)
    indices = out16[:, :7].astype(jnp.int32)
    pscores = out16[:, 8:15]
    wscores = jnp.concatenate(
        [scores_pad[:, _G_BASE[g]:_G_BASE[g] + _G_SIZES[g]] for g in range(3)],
        axis=1)
    return (indices, pscores, wscores)


def kernel(proposalN, x, ratios, window_nums_sum, N_list, iou_threshs):
    return _impl(proposalN, x, ratios, window_nums_sum, N_list, iou_threshs)
